# baseline (speedup 1.0000x reference)
import functools

import jax
import jax.numpy as jnp
from jax import lax
from jax.experimental import pallas as pl
from jax.experimental.pallas import tpu as pltpu
from jax.experimental.pallas import tpu_sc as plsc

MAX_POS = 1024
HIDDEN = 768
EPS = 1e-12

NC = 2
NS = 16
NW = NC * NS

TC_CHUNK = 32
G_CHUNK = 4 * TC_CHUNK
HW2 = HIDDEN // 2
LVECS = HW2 // 16


def _sc_gather_sum(tbl_w, idx_grouped, n_tok):
    tok_per_w = n_tok // NW
    n_chunks = tok_per_w // TC_CHUNK
    mesh = plsc.VectorSubcoreMesh(core_axis_name="c", subcore_axis_name="s")

    @functools.partial(
        pl.kernel,
        out_type=jax.ShapeDtypeStruct((n_tok, HW2), jnp.float32),
        mesh=mesh,
        scratch_types=[
            pltpu.VMEM((G_CHUNK * n_chunks,), jnp.int32),
            pltpu.VMEM((G_CHUNK, HW2), jnp.float32),
            pltpu.VMEM((G_CHUNK, HW2), jnp.float32),
            pltpu.VMEM((TC_CHUNK, HW2), jnp.float32),
            pltpu.VMEM((TC_CHUNK, HW2), jnp.float32),
            pltpu.SemaphoreType.DMA,
            pltpu.SemaphoreType.DMA,
            pltpu.SemaphoreType.DMA,
            pltpu.SemaphoreType.DMA,
        ],
    )
    def k(tbl_hbm, idx_hbm, out_hbm, idx_full, rows0, rows1, acc0, acc1,
          sem0, sem1, osem0, osem1):
        wid = lax.axis_index("s") * NC + lax.axis_index("c")
        ibase = wid * (tok_per_w * 4)
        obase = wid * tok_per_w
        rows_v = (rows0, rows1)
        acc_v = (acc0, acc1)
        sems = (sem0, sem1)
        osems = (osem0, osem1)
        copies = [None, None]
        stores = [None, None]
        pltpu.sync_copy(idx_hbm.at[pl.ds(ibase, G_CHUNK * n_chunks)], idx_full)
        copies[0] = pltpu.async_copy(
            tbl_hbm.at[idx_full.at[pl.ds(0, G_CHUNK)]], rows0, sem0
        )
        for c in range(n_chunks):
            cur = c % 2
            nxt = (c + 1) % 2
            if c + 1 < n_chunks:
                copies[nxt] = pltpu.async_copy(
                    tbl_hbm.at[idx_full.at[pl.ds((c + 1) * G_CHUNK, G_CHUNK)]],
                    rows_v[nxt], sems[nxt]
                )
            copies[cur].wait()
            if stores[cur] is not None:
                stores[cur].wait()
            r = rows_v[cur]
            a = acc_v[cur]

            @plsc.parallel_loop(0, TC_CHUNK, 1, unroll=1)
            def body(t, r=r, a=a):
                msk = jnp.uint32(0xFFFF0000)
                for v in range(LVECS):
                    sl = pl.ds(v * 16, 16)
                    f0 = r[t, sl]
                    f1 = r[TC_CHUNK + t, sl]
                    f2 = r[2 * TC_CHUNK + t, sl]
                    f3 = r[3 * TC_CHUNK + t, sl]
                    hi = (f0 + f1) + (f2 + f3)
                    lo = (
                        lax.bitcast_convert_type(
                            lax.bitcast_convert_type(f0, jnp.uint32) << 16,
                            jnp.float32)
                        + lax.bitcast_convert_type(
                            lax.bitcast_convert_type(f1, jnp.uint32) << 16,
                            jnp.float32)
                    ) + (
                        lax.bitcast_convert_type(
                            lax.bitcast_convert_type(f2, jnp.uint32) << 16,
                            jnp.float32)
                        + lax.bitcast_convert_type(
                            lax.bitcast_convert_type(f3, jnp.uint32) << 16,
                            jnp.float32)
                    )
                    packed = (lax.bitcast_convert_type(lo, jnp.uint32) >> 16) | (
                        lax.bitcast_convert_type(hi, jnp.uint32) & msk
                    )
                    a[t, sl] = lax.bitcast_convert_type(packed, jnp.float32)

            stores[cur] = pltpu.async_copy(
                a, out_hbm.at[pl.ds(obase + c * TC_CHUNK, TC_CHUNK)], osems[cur]
            )
        for st in stores:
            if st is not None:
                st.wait()

    return k(tbl_w, idx_grouped)


BT = 2048


def _tc_body(emb_ref, g_ref, bt_ref, wt_ref, b_ref, out_ref):
    w = lax.bitcast_convert_type(emb_ref[...], jnp.uint32)
    lo = lax.bitcast_convert_type(w << 16, jnp.float32)
    hi = lax.bitcast_convert_type(w & jnp.uint32(0xFFFF0000), jnp.float32)
    emb = jnp.concatenate([lo, hi], axis=-1)
    mean = jnp.mean(emb, axis=-1, keepdims=True)
    d = emb - mean
    var = jnp.mean(d * d, axis=-1, keepdims=True)
    nrm = d * lax.rsqrt(var + EPS) * g_ref[...] + bt_ref[...]
    out_ref[...] = (
        jnp.dot(
            nrm.astype(jnp.bfloat16), wt_ref[...],
            preferred_element_type=jnp.float32,
        )
        + b_ref[...]
    )


def _tc_ln_mlp(emb_w, gamma, beta, w_t, b):
    n_tok = emb_w.shape[0]
    return pl.pallas_call(
        _tc_body,
        grid=(n_tok // BT,),
        in_specs=[
            pl.BlockSpec((BT, HW2), lambda i: (i, 0)),
            pl.BlockSpec((1, HIDDEN), lambda i: (0, 0)),
            pl.BlockSpec((1, HIDDEN), lambda i: (0, 0)),
            pl.BlockSpec((HIDDEN, HIDDEN), lambda i: (0, 0)),
            pl.BlockSpec((1, HIDDEN), lambda i: (0, 0)),
        ],
        out_specs=pl.BlockSpec((BT, HIDDEN), lambda i: (i, 0)),
        out_shape=jax.ShapeDtypeStruct((n_tok, HIDDEN), jnp.float32),
    )(emb_w, gamma, beta, w_t, b)


N_SLICES = 1


def kernel(bbox, x_table, y_table, ln_gamma, ln_beta, W, b):
    batch, seq, _ = bbox.shape
    n_tok = batch * seq
    idx = bbox.reshape(n_tok, 4).astype(jnp.int32)
    comps = jnp.stack(
        [idx[:, 0], idx[:, 1] + MAX_POS, idx[:, 2], idx[:, 3] + MAX_POS], axis=0
    )
    tbl = jnp.concatenate([x_table, y_table], axis=0).astype(jnp.bfloat16)
    bits = lax.bitcast_convert_type(tbl, jnp.uint16).astype(jnp.uint32)
    words = bits[:, :HW2] | (bits[:, HW2:] << 16)
    tbl_w = lax.bitcast_convert_type(words, jnp.float32)
    gamma = ln_gamma.reshape(1, HIDDEN)
    beta = ln_beta.reshape(1, HIDDEN)
    w_t = W.T.astype(jnp.bfloat16)
    bias = b.reshape(1, HIDDEN)
    n_slc = n_tok // N_SLICES
    outs = []
    for s in range(N_SLICES):
        cs = comps[:, s * n_slc:(s + 1) * n_slc]
        idx_grouped = (
            cs.reshape(4, NW, n_slc // (NW * TC_CHUNK), TC_CHUNK)
            .transpose(1, 2, 0, 3)
            .reshape(-1)
        )
        emb_w = _sc_gather_sum(tbl_w, idx_grouped, n_slc)
        outs.append(_tc_ln_mlp(emb_w, gamma, beta, w_t, bias))
    out = jnp.concatenate(outs, axis=0)
    return out.reshape(batch, seq, HIDDEN)


# speedup vs baseline: 1.0090x; 1.0090x over previous
import functools

import jax
import jax.numpy as jnp
from jax import lax
from jax.experimental import pallas as pl
from jax.experimental.pallas import tpu as pltpu
from jax.experimental.pallas import tpu_sc as plsc

MAX_POS = 1024
HIDDEN = 768
EPS = 1e-12

NC = 2
NS = 16
NW = NC * NS

TC_CHUNK = 32
G_CHUNK = 4 * TC_CHUNK
HW2 = HIDDEN // 2
LVECS = HW2 // 16


def _sc_gather_sum(tbl_w, idx_grouped, n_tok):
    tok_per_w = n_tok // NW
    n_chunks = tok_per_w // TC_CHUNK
    mesh = plsc.VectorSubcoreMesh(core_axis_name="c", subcore_axis_name="s")

    @functools.partial(
        pl.kernel,
        out_type=jax.ShapeDtypeStruct((n_tok, HW2), jnp.float32),
        mesh=mesh,
        scratch_types=[
            pltpu.VMEM((G_CHUNK,), jnp.int32),
            pltpu.VMEM((G_CHUNK,), jnp.int32),
            pltpu.VMEM((G_CHUNK, HW2), jnp.float32),
            pltpu.VMEM((G_CHUNK, HW2), jnp.float32),
            pltpu.VMEM((TC_CHUNK, HW2), jnp.float32),
            pltpu.VMEM((TC_CHUNK, HW2), jnp.float32),
            pltpu.SemaphoreType.DMA,
            pltpu.SemaphoreType.DMA,
            pltpu.SemaphoreType.DMA,
            pltpu.SemaphoreType.DMA,
        ],
    )
    def k(tbl_hbm, idx_hbm, out_hbm, idx0, idx1, rows0, rows1, acc0, acc1,
          sem0, sem1, osem0, osem1):
        wid = lax.axis_index("s") * NC + lax.axis_index("c")
        ibase = wid * (tok_per_w * 4)
        obase = wid * tok_per_w
        idx_v = (idx0, idx1)
        rows_v = (rows0, rows1)
        acc_v = (acc0, acc1)
        sems = (sem0, sem1)
        osems = (osem0, osem1)
        copies = [None, None]
        stores = [None, None]
        pltpu.sync_copy(idx_hbm.at[pl.ds(ibase, G_CHUNK)], idx0)
        copies[0] = pltpu.async_copy(tbl_hbm.at[idx0], rows0, sem0)
        for c in range(n_chunks):
            cur = c % 2
            nxt = (c + 1) % 2
            if c + 1 < n_chunks:
                off_n = ibase + (c + 1) * G_CHUNK
                pltpu.sync_copy(idx_hbm.at[pl.ds(off_n, G_CHUNK)], idx_v[nxt])
                copies[nxt] = pltpu.async_copy(
                    tbl_hbm.at[idx_v[nxt]], rows_v[nxt], sems[nxt]
                )
            copies[cur].wait()
            if stores[cur] is not None:
                stores[cur].wait()
            r = rows_v[cur]
            a = acc_v[cur]

            @plsc.parallel_loop(0, TC_CHUNK, 1, unroll=1)
            def body(t, r=r, a=a):
                msk = jnp.uint32(0xFFFF0000)
                for v in range(LVECS):
                    sl = pl.ds(v * 16, 16)
                    f0 = r[t, sl]
                    f1 = r[TC_CHUNK + t, sl]
                    f2 = r[2 * TC_CHUNK + t, sl]
                    f3 = r[3 * TC_CHUNK + t, sl]
                    hi = (f0 + f1) + (f2 + f3)
                    lo = (
                        lax.bitcast_convert_type(
                            lax.bitcast_convert_type(f0, jnp.uint32) << 16,
                            jnp.float32)
                        + lax.bitcast_convert_type(
                            lax.bitcast_convert_type(f1, jnp.uint32) << 16,
                            jnp.float32)
                    ) + (
                        lax.bitcast_convert_type(
                            lax.bitcast_convert_type(f2, jnp.uint32) << 16,
                            jnp.float32)
                        + lax.bitcast_convert_type(
                            lax.bitcast_convert_type(f3, jnp.uint32) << 16,
                            jnp.float32)
                    )
                    packed = (lax.bitcast_convert_type(lo, jnp.uint32) >> 16) | (
                        lax.bitcast_convert_type(hi, jnp.uint32) & msk
                    )
                    a[t, sl] = lax.bitcast_convert_type(packed, jnp.float32)

            stores[cur] = pltpu.async_copy(
                a, out_hbm.at[pl.ds(obase + c * TC_CHUNK, TC_CHUNK)], osems[cur]
            )
        for st in stores:
            if st is not None:
                st.wait()

    return k(tbl_w, idx_grouped)


BT = 2048


def _tc_body(emb_ref, g_ref, bt_ref, wt_ref, b_ref, out_ref):
    w = lax.bitcast_convert_type(emb_ref[...], jnp.uint32)
    lo = lax.bitcast_convert_type(w << 16, jnp.float32)
    hi = lax.bitcast_convert_type(w & jnp.uint32(0xFFFF0000), jnp.float32)
    emb = jnp.concatenate([lo, hi], axis=-1)
    mean = jnp.mean(emb, axis=-1, keepdims=True)
    d = emb - mean
    var = jnp.mean(d * d, axis=-1, keepdims=True)
    nrm = d * lax.rsqrt(var + EPS) * g_ref[...] + bt_ref[...]
    out_ref[...] = (
        jnp.dot(
            nrm.astype(jnp.bfloat16), wt_ref[...],
            preferred_element_type=jnp.float32,
        )
        + b_ref[...]
    )


def _tc_ln_mlp(emb_w, gamma, beta, w_t, b):
    n_tok = emb_w.shape[0]
    return pl.pallas_call(
        _tc_body,
        grid=(n_tok // BT,),
        in_specs=[
            pl.BlockSpec((BT, HW2), lambda i: (i, 0)),
            pl.BlockSpec((1, HIDDEN), lambda i: (0, 0)),
            pl.BlockSpec((1, HIDDEN), lambda i: (0, 0)),
            pl.BlockSpec((HIDDEN, HIDDEN), lambda i: (0, 0)),
            pl.BlockSpec((1, HIDDEN), lambda i: (0, 0)),
        ],
        out_specs=pl.BlockSpec((BT, HIDDEN), lambda i: (i, 0)),
        out_shape=jax.ShapeDtypeStruct((n_tok, HIDDEN), jnp.float32),
    )(emb_w, gamma, beta, w_t, b)


N_SLICES = 1


def kernel(bbox, x_table, y_table, ln_gamma, ln_beta, W, b):
    batch, seq, _ = bbox.shape
    n_tok = batch * seq
    idx = bbox.reshape(n_tok, 4).astype(jnp.int32)
    comps = jnp.stack(
        [idx[:, 0], idx[:, 1] + MAX_POS, idx[:, 2], idx[:, 3] + MAX_POS], axis=0
    )
    tbl = jnp.concatenate([x_table, y_table], axis=0).astype(jnp.bfloat16)
    bits = lax.bitcast_convert_type(tbl, jnp.uint16).astype(jnp.uint32)
    words = bits[:, :HW2] | (bits[:, HW2:] << 16)
    tbl_w = lax.bitcast_convert_type(words, jnp.float32)
    gamma = ln_gamma.reshape(1, HIDDEN)
    beta = ln_beta.reshape(1, HIDDEN)
    w_t = W.T.astype(jnp.bfloat16)
    bias = b.reshape(1, HIDDEN)
    n_slc = n_tok // N_SLICES
    outs = []
    for s in range(N_SLICES):
        cs = comps[:, s * n_slc:(s + 1) * n_slc]
        idx_grouped = (
            cs.reshape(4, NW, n_slc // (NW * TC_CHUNK), TC_CHUNK)
            .transpose(1, 2, 0, 3)
            .reshape(-1)
        )
        emb_w = _sc_gather_sum(tbl_w, idx_grouped, n_slc)
        outs.append(_tc_ln_mlp(emb_w, gamma, beta, w_t, bias))
    out = jnp.concatenate(outs, axis=0)
    return out.reshape(batch, seq, HIDDEN)
